# baseline (speedup 1.0000x reference)
import functools

import numpy as np
import jax
import jax.numpy as jnp
from jax import lax
from jax.experimental import pallas as pl
from jax.experimental.pallas import tpu as pltpu
from jax.experimental.pallas import tpu_sc as plsc

NUM_LEVELS = 16
MAX_ENTRIES = 2 ** 14
_ln512 = np.log(np.float32(512.0))
_ln16 = np.log(np.float32(16.0))
_GROW = float(np.exp((_ln512 - _ln16) / np.float32(15.0)))
NS = [int(16 * _GROW ** i) for i in range(NUM_LEVELS)]
_P1 = np.int64(2654435761).astype(np.int32)
_P2 = np.int64(805459861).astype(np.int32)

_CONSTS = np.zeros((5, NUM_LEVELS), dtype=np.int32)
for _l, _n in enumerate(NS):
    _dense = (_n + 1) ** 3 <= MAX_ENTRIES
    _CONSTS[0, _l] = _n
    _CONSTS[1, _l] = (_n + 1) ** 2 if _dense else 1
    _CONSTS[2, _l] = (_n + 1) if _dense else _P1
    _CONSTS[3, _l] = 1 if _dense else _P2
    _CONSTS[4, _l] = 1 if _dense else 0

B_TOTAL = 262144
B_HALF = B_TOTAL // 2
CHUNK = 4096
N_CHUNKS = B_HALF // CHUNK
GROUPS = CHUNK // 16


def _encode_sc(xT, tabs, consts):
    mesh = plsc.VectorSubcoreMesh(core_axis_name="c", subcore_axis_name="s")

    @functools.partial(
        pl.kernel,
        mesh=mesh,
        compiler_params=pltpu.CompilerParams(needs_layout_passes=False),
        out_type=jax.ShapeDtypeStruct((2 * NUM_LEVELS, B_TOTAL), jnp.float32),
        scratch_types=[
            pltpu.VMEM((2 * MAX_ENTRIES,), jnp.float32),
            pltpu.VMEM((2, 3, CHUNK), jnp.float32),
            pltpu.VMEM((2, 2, CHUNK), jnp.float32),
            pltpu.VMEM((5 * NUM_LEVELS,), jnp.int32),
            pltpu.SemaphoreType.DMA,
            pltpu.SemaphoreType.DMA,
            pltpu.SemaphoreType.DMA,
            pltpu.SemaphoreType.DMA,
        ],
    )
    def enc_kernel(xT_hbm, tabs_hbm, consts_hbm, out_hbm,
                   table_v, coords_v, feat_v, consts_v,
                   sem_in0, sem_in1, sem_out0, sem_out1):
        level = lax.axis_index("s").astype(jnp.int32)
        half = lax.axis_index("c").astype(jnp.int32)

        pltpu.sync_copy(consts_hbm, consts_v)
        pltpu.sync_copy(tabs_hbm.at[level], table_v)

        lane_l = jnp.zeros((16,), jnp.int32) + level

        def splat(row):
            ridx = jnp.full((16,), row * NUM_LEVELS, dtype=jnp.int32)
            return plsc.load_gather(consts_v, [ridx + lane_l])

        nl_f = splat(0).astype(jnp.float32)
        c0 = splat(1)
        c1 = splat(2)
        c2 = splat(3)
        dense = splat(4) == 1

        i32 = jnp.int32
        base_half = half * i32(B_HALF)
        sem_in = (sem_in0, sem_in1)
        sem_out = (sem_out0, sem_out1)

        def in_copy(ci, b):
            base = base_half + ci * i32(CHUNK)
            return pltpu.make_async_copy(
                xT_hbm.at[pl.ds(0, 3), pl.ds(base, CHUNK)],
                coords_v.at[i32(b)], sem_in[b])

        def out_copy(ci, b):
            base = base_half + ci * i32(CHUNK)
            return pltpu.make_async_copy(
                feat_v.at[i32(b)],
                out_hbm.at[pl.ds(2 * level, 2), pl.ds(base, CHUNK)],
                sem_out[b])

        in_copy(i32(0), 0).start()

        def chunk_pair_body(pi, carry):
            for b in (0, 1):
                ci = pi * i32(2) + i32(b)
                in_copy(ci, b).wait()

                @pl.when(ci + i32(1) < i32(N_CHUNKS))
                def _():
                    in_copy(ci + i32(1), 1 - b).start()

                @pl.when(ci >= i32(2))
                def _():
                    out_copy(ci - i32(2), b).wait()

                compute_chunk(b)
                out_copy(ci, b).start()
            return carry

        def compute_chunk(b):
            def group_body(g):
                o = g * i32(16)
                dims = []
                for d in range(3):
                    xg = coords_v[b, d, pl.ds(o, 16)]
                    sc = xg * 0.1 + 0.5
                    sc = jnp.minimum(jnp.maximum(sc, 0.0), 1.0 - 1e-6)
                    xl = sc * nl_f
                    fi = xl.astype(jnp.int32)
                    w = xl - fi.astype(jnp.float32)
                    dims.append((fi, w))
                (fi0, w0), (fi1, w1), (fi2, w2) = dims
                a0 = (fi0 * c0, fi0 * c0 + c0)
                a1 = (fi1 * c1, fi1 * c1 + c1)
                a2 = (fi2 * c2, fi2 * c2 + c2)
                wx = (1.0 - w0, w0)
                wy = (1.0 - w1, w1)
                wz = (1.0 - w2, w2)
                wxy = {(i, j): wx[i] * wy[j] for i in (0, 1) for j in (0, 1)}
                acc0 = jnp.zeros((16,), jnp.float32)
                acc1 = jnp.zeros((16,), jnp.float32)
                for v in range(8):
                    b0, b1, b2 = (v >> 2) & 1, (v >> 1) & 1, v & 1
                    hidx = (a0[b0] ^ a1[b1] ^ a2[b2]) & (MAX_ENTRIES - 1)
                    didx = a0[b0] + a1[b1] + a2[b2]
                    idx = jnp.where(dense, didx, hidx)
                    f0 = idx * 2
                    g0 = plsc.load_gather(table_v, [f0])
                    g1 = plsc.load_gather(table_v, [f0 + 1])
                    vw = wxy[(b0, b1)] * wz[b2]
                    acc0 = acc0 + g0 * vw
                    acc1 = acc1 + g1 * vw
                feat_v[b, 0, pl.ds(o, 16)] = acc0
                feat_v[b, 1, pl.ds(o, 16)] = acc1

            def wrapped(q, carry2):
                g4 = q * i32(4)
                for k in range(4):
                    group_body(g4 + i32(k))
                return carry2
            lax.fori_loop(i32(0), i32(GROUPS // 4), wrapped, i32(0))

        lax.fori_loop(i32(0), i32(N_CHUNKS // 2), chunk_pair_body, i32(0))
        out_copy(i32(N_CHUNKS - 2), 0).wait()
        out_copy(i32(N_CHUNKS - 1), 1).wait()

    return enc_kernel(xT, tabs, consts)


_BB = 8192


def _mlp_body(enc_ref, xv_ref, wd0, bd0, wd1, bd1, wc0v, wc0h, bc0,
              wc1, bc1, wc2, bc2, out_ref):
    enc = enc_ref[...]
    mm = functools.partial(jnp.dot, preferred_element_type=jnp.float32)
    h = jnp.maximum(mm(wd0[...], enc) + bd0[...], 0.0)
    h = mm(wd1[...], h) + bd1[...]
    row = lax.broadcasted_iota(jnp.int32, (16, _BB), 0)
    hh = jnp.where(row == 0, jax.nn.sigmoid(h), jnp.maximum(h, 0.0))
    hc = mm(wc0h[...], hh) + mm(wc0v[...], xv_ref[...]) + bc0[...]
    hc = jnp.maximum(hc, 0.0)
    hc = jnp.maximum(mm(wc1[...], hc) + bc1[...], 0.0)
    hc = mm(wc2[...], hc) + bc2[...]
    out_ref[...] = jnp.concatenate(
        [hh[0:1, :], jax.nn.sigmoid(hc)], axis=0)


def _mlp_tc(enc, xvT, wd0T, bd0, wd1T, bd1, wc0vT, wc0hT, bc0,
            wc1T, bc1, wc2T, bc2):
    grid = B_TOTAL // _BB
    z = np.int32(0)
    full = lambda shape: pl.BlockSpec(shape, lambda i: (z, z))
    batch = lambda rows: pl.BlockSpec((rows, _BB), lambda i: (z, i))
    return pl.pallas_call(
        _mlp_body,
        grid=(grid,),
        in_specs=[
            batch(32),
            batch(3),
            full((64, 32)), full((64, 1)),
            full((16, 64)), full((16, 1)),
            full((64, 3)), full((64, 16)), full((64, 1)),
            full((64, 64)), full((64, 1)),
            full((3, 64)), full((3, 1)),
        ],
        out_specs=batch(4),
        out_shape=jax.ShapeDtypeStruct((4, B_TOTAL), jnp.float32),
    )(enc, xvT, wd0T, bd0, wd1T, bd1, wc0vT, wc0hT, bc0,
      wc1T, bc1, wc2T, bc2)


def kernel(x, tables, w_d0, b_d0, w_d1, b_d1, w_c0, b_c0, w_c1, b_c1,
           w_c2, b_c2):
    x = x.astype(jnp.float32)
    xT = x.T
    tabs = tables.astype(jnp.float32).reshape(NUM_LEVELS, 2 * MAX_ENTRIES)
    consts = jnp.asarray(_CONSTS.reshape(-1))

    enc = _encode_sc(xT[0:3], tabs, consts)

    outT = _mlp_tc(
        enc, xT[3:6],
        w_d0.T, b_d0.reshape(64, 1),
        w_d1.T, b_d1.reshape(16, 1),
        w_c0[16:19].T, w_c0[0:16].T, b_c0.reshape(64, 1),
        w_c1.T, b_c1.reshape(64, 1),
        w_c2.T, b_c2.reshape(3, 1),
    )
    return outT.T


# speedup vs baseline: 3.5948x; 3.5948x over previous
import functools

import numpy as np
import jax
import jax.numpy as jnp
from jax import lax
from jax.experimental import pallas as pl
from jax.experimental.pallas import tpu as pltpu
from jax.experimental.pallas import tpu_sc as plsc

NUM_LEVELS = 16
MAX_ENTRIES = 2 ** 14
_ln512 = np.log(np.float32(512.0))
_ln16 = np.log(np.float32(16.0))
_GROW = float(np.exp((_ln512 - _ln16) / np.float32(15.0)))
NS = [int(16 * _GROW ** i) for i in range(NUM_LEVELS)]
_P1 = np.int64(2654435761).astype(np.int32)
_P2 = np.int64(805459861).astype(np.int32)

_CONSTS = np.zeros((5, NUM_LEVELS), dtype=np.int32)
for _l, _n in enumerate(NS):
    _dense = (_n + 1) ** 3 <= MAX_ENTRIES
    _CONSTS[0, _l] = _n
    _CONSTS[1, _l] = (_n + 1) ** 2 if _dense else 1
    _CONSTS[2, _l] = (_n + 1) if _dense else _P1
    _CONSTS[3, _l] = 1 if _dense else _P2
    _CONSTS[4, _l] = 1 if _dense else 0

B_TOTAL = 262144
B_HALF = B_TOTAL // 2
CHUNK = 4096
N_CHUNKS = B_HALF // CHUNK
GROUPS = CHUNK // 16


def _encode_sc(xT, tabs, consts):
    mesh = plsc.VectorSubcoreMesh(core_axis_name="c", subcore_axis_name="s")

    @functools.partial(
        pl.kernel,
        mesh=mesh,
        compiler_params=pltpu.CompilerParams(needs_layout_passes=False),
        out_type=jax.ShapeDtypeStruct((2 * NUM_LEVELS, B_TOTAL), jnp.float32),
        scratch_types=[
            pltpu.VMEM((2 * MAX_ENTRIES,), jnp.float32),
            pltpu.VMEM((2, 3, CHUNK), jnp.float32),
            pltpu.VMEM((2, 2, CHUNK), jnp.float32),
            pltpu.VMEM((5 * NUM_LEVELS,), jnp.int32),
            pltpu.SemaphoreType.DMA,
            pltpu.SemaphoreType.DMA,
            pltpu.SemaphoreType.DMA,
            pltpu.SemaphoreType.DMA,
        ],
    )
    def enc_kernel(xT_hbm, tabs_hbm, consts_hbm, out_hbm,
                   table_v, coords_v, feat_v, consts_v,
                   sem_in0, sem_in1, sem_out0, sem_out1):
        level = lax.axis_index("s").astype(jnp.int32)
        half = lax.axis_index("c").astype(jnp.int32)

        pltpu.sync_copy(consts_hbm, consts_v)
        pltpu.sync_copy(tabs_hbm.at[level], table_v)

        lane_l = jnp.zeros((16,), jnp.int32) + level

        def splat(row):
            ridx = jnp.full((16,), row * NUM_LEVELS, dtype=jnp.int32)
            return plsc.load_gather(consts_v, [ridx + lane_l])

        nl_f = splat(0).astype(jnp.float32)
        c0 = splat(1)
        c1 = splat(2)
        c2 = splat(3)
        dense = splat(4) == 1

        i32 = jnp.int32
        base_half = half * i32(B_HALF)
        sem_in = (sem_in0, sem_in1)
        sem_out = (sem_out0, sem_out1)

        def in_copy(ci, b):
            base = base_half + ci * i32(CHUNK)
            return pltpu.make_async_copy(
                xT_hbm.at[pl.ds(0, 3), pl.ds(base, CHUNK)],
                coords_v.at[i32(b)], sem_in[b])

        def out_copy(ci, b):
            base = base_half + ci * i32(CHUNK)
            return pltpu.make_async_copy(
                feat_v.at[i32(b)],
                out_hbm.at[pl.ds(2 * level, 2), pl.ds(base, CHUNK)],
                sem_out[b])

        def chunk_pair_body(pi, carry):
            for b in (0, 1):
                ci = pi * i32(2) + i32(b)
                in_copy(ci, b).start()
                in_copy(ci, b).wait()
                compute_chunk(b)
                out_copy(ci, b).start()
                out_copy(ci, b).wait()
            return carry

        def compute_chunk(b):
            @functools.partial(
                plsc.parallel_loop, i32(0), i32(GROUPS), unroll=2)
            def group_body(g):
                o = g * i32(16)
                dims = []
                for d in range(3):
                    xg = coords_v[b, d, pl.ds(o, 16)]
                    sc = xg * 0.1 + 0.5
                    sc = jnp.minimum(jnp.maximum(sc, 0.0), 1.0 - 1e-6)
                    xl = sc * nl_f
                    fi = xl.astype(jnp.int32)
                    w = xl - fi.astype(jnp.float32)
                    dims.append((fi, w))
                (fi0, w0), (fi1, w1), (fi2, w2) = dims
                a0 = (fi0 * c0, fi0 * c0 + c0)
                a1 = (fi1 * c1, fi1 * c1 + c1)
                a2 = (fi2 * c2, fi2 * c2 + c2)
                wx = (1.0 - w0, w0)
                wy = (1.0 - w1, w1)
                wz = (1.0 - w2, w2)
                wxy = {(i, j): wx[i] * wy[j] for i in (0, 1) for j in (0, 1)}
                acc0 = jnp.zeros((16,), jnp.float32)
                acc1 = jnp.zeros((16,), jnp.float32)
                for v in range(8):
                    b0, b1, b2 = (v >> 2) & 1, (v >> 1) & 1, v & 1
                    hidx = (a0[b0] ^ a1[b1] ^ a2[b2]) & (MAX_ENTRIES - 1)
                    didx = a0[b0] + a1[b1] + a2[b2]
                    idx = jnp.where(dense, didx, hidx)
                    f0 = idx * 2
                    g0 = plsc.load_gather(table_v, [f0])
                    g1 = plsc.load_gather(table_v, [f0 + 1])
                    vw = wxy[(b0, b1)] * wz[b2]
                    acc0 = acc0 + g0 * vw
                    acc1 = acc1 + g1 * vw
                feat_v[b, 0, pl.ds(o, 16)] = acc0
                feat_v[b, 1, pl.ds(o, 16)] = acc1

        lax.fori_loop(i32(0), i32(N_CHUNKS // 2), chunk_pair_body, i32(0))

    return enc_kernel(xT, tabs, consts)


_BB = 8192


def _mlp_body(enc_ref, xv_ref, wd0, bd0, wd1, bd1, wc0v, wc0h, bc0,
              wc1, bc1, wc2, bc2, out_ref):
    enc = enc_ref[...]
    mm = functools.partial(jnp.dot, preferred_element_type=jnp.float32)
    h = jnp.maximum(mm(wd0[...], enc) + bd0[...], 0.0)
    h = mm(wd1[...], h) + bd1[...]
    row = lax.broadcasted_iota(jnp.int32, (16, _BB), 0)
    hh = jnp.where(row == 0, jax.nn.sigmoid(h), jnp.maximum(h, 0.0))
    hc = mm(wc0h[...], hh) + mm(wc0v[...], xv_ref[...]) + bc0[...]
    hc = jnp.maximum(hc, 0.0)
    hc = jnp.maximum(mm(wc1[...], hc) + bc1[...], 0.0)
    hc = mm(wc2[...], hc) + bc2[...]
    out_ref[...] = jnp.concatenate(
        [hh[0:1, :], jax.nn.sigmoid(hc)], axis=0)


def _mlp_tc(enc, xvT, wd0T, bd0, wd1T, bd1, wc0vT, wc0hT, bc0,
            wc1T, bc1, wc2T, bc2):
    grid = B_TOTAL // _BB
    z = np.int32(0)
    full = lambda shape: pl.BlockSpec(shape, lambda i: (z, z))
    batch = lambda rows: pl.BlockSpec((rows, _BB), lambda i: (z, i))
    return pl.pallas_call(
        _mlp_body,
        grid=(grid,),
        in_specs=[
            batch(32),
            batch(3),
            full((64, 32)), full((64, 1)),
            full((16, 64)), full((16, 1)),
            full((64, 3)), full((64, 16)), full((64, 1)),
            full((64, 64)), full((64, 1)),
            full((3, 64)), full((3, 1)),
        ],
        out_specs=batch(4),
        out_shape=jax.ShapeDtypeStruct((4, B_TOTAL), jnp.float32),
    )(enc, xvT, wd0T, bd0, wd1T, bd1, wc0vT, wc0hT, bc0,
      wc1T, bc1, wc2T, bc2)


def kernel(x, tables, w_d0, b_d0, w_d1, b_d1, w_c0, b_c0, w_c1, b_c1,
           w_c2, b_c2):
    x = x.astype(jnp.float32)
    xT = x.T
    tabs = tables.astype(jnp.float32).reshape(NUM_LEVELS, 2 * MAX_ENTRIES)
    consts = jnp.asarray(_CONSTS.reshape(-1))

    enc = _encode_sc(xT[0:3], tabs, consts)

    outT = _mlp_tc(
        enc, xT[3:6],
        w_d0.T, b_d0.reshape(64, 1),
        w_d1.T, b_d1.reshape(16, 1),
        w_c0[16:19].T, w_c0[0:16].T, b_c0.reshape(64, 1),
        w_c1.T, b_c1.reshape(64, 1),
        w_c2.T, b_c2.reshape(3, 1),
    )
    return outT.T


# speedup vs baseline: 3.7439x; 1.0415x over previous
import functools

import numpy as np
import jax
import jax.numpy as jnp
from jax import lax
from jax.experimental import pallas as pl
from jax.experimental.pallas import tpu as pltpu
from jax.experimental.pallas import tpu_sc as plsc

NUM_LEVELS = 16
MAX_ENTRIES = 2 ** 14
_ln512 = np.log(np.float32(512.0))
_ln16 = np.log(np.float32(16.0))
_GROW = float(np.exp((_ln512 - _ln16) / np.float32(15.0)))
NS = [int(16 * _GROW ** i) for i in range(NUM_LEVELS)]
_P1 = np.int64(2654435761).astype(np.int32)
_P2 = np.int64(805459861).astype(np.int32)

_CONSTS = np.zeros((5, NUM_LEVELS), dtype=np.int32)
for _l, _n in enumerate(NS):
    _dense = (_n + 1) ** 3 <= MAX_ENTRIES
    _CONSTS[0, _l] = _n
    _CONSTS[1, _l] = (_n + 1) ** 2 if _dense else 1
    _CONSTS[2, _l] = (_n + 1) if _dense else _P1
    _CONSTS[3, _l] = 1 if _dense else _P2
    _CONSTS[4, _l] = 1 if _dense else 0

B_TOTAL = 262144
B_HALF = B_TOTAL // 2
CHUNK = 4096
N_CHUNKS = B_HALF // CHUNK
GROUPS = CHUNK // 16


def _encode_sc(xT, tabs, consts):
    mesh = plsc.VectorSubcoreMesh(core_axis_name="c", subcore_axis_name="s")

    @functools.partial(
        pl.kernel,
        mesh=mesh,
        compiler_params=pltpu.CompilerParams(needs_layout_passes=False),
        out_type=jax.ShapeDtypeStruct((2 * NUM_LEVELS, B_TOTAL), jnp.float32),
        scratch_types=[
            pltpu.VMEM((2 * MAX_ENTRIES,), jnp.float32),
            pltpu.VMEM((2, 3, CHUNK), jnp.float32),
            pltpu.VMEM((2, 2, CHUNK), jnp.float32),
            pltpu.VMEM((5 * NUM_LEVELS,), jnp.int32),
            pltpu.SemaphoreType.DMA,
            pltpu.SemaphoreType.DMA,
            pltpu.SemaphoreType.DMA,
            pltpu.SemaphoreType.DMA,
        ],
    )
    def enc_kernel(xT_hbm, tabs_hbm, consts_hbm, out_hbm,
                   table_v, coords_v, feat_v, consts_v,
                   sem_in0, sem_in1, sem_out0, sem_out1):
        level = lax.axis_index("s").astype(jnp.int32)
        half = lax.axis_index("c").astype(jnp.int32)

        pltpu.sync_copy(consts_hbm, consts_v)
        pltpu.sync_copy(tabs_hbm.at[level], table_v)

        lane_l = jnp.zeros((16,), jnp.int32) + level

        def splat(row):
            ridx = jnp.full((16,), row * NUM_LEVELS, dtype=jnp.int32)
            return plsc.load_gather(consts_v, [ridx + lane_l])

        nl_f = splat(0).astype(jnp.float32)
        c0 = splat(1)
        c1 = splat(2)
        c2 = splat(3)
        dense = splat(4) == 1

        i32 = jnp.int32
        base_half = half * i32(B_HALF)
        sem_in = (sem_in0, sem_in1)
        sem_out = (sem_out0, sem_out1)

        def in_copy(ci, b):
            base = base_half + ci * i32(CHUNK)
            return pltpu.make_async_copy(
                xT_hbm.at[pl.ds(0, 3), pl.ds(base, CHUNK)],
                coords_v.at[i32(b)], sem_in[b])

        def out_copy(ci, b, dep=None):
            base = base_half + ci * i32(CHUNK)
            if dep is not None:
                base = pl.multiple_of(base + dep, CHUNK)
            return pltpu.make_async_copy(
                feat_v.at[i32(b)],
                out_hbm.at[pl.ds(2 * level, 2), pl.ds(base, CHUNK)],
                sem_out[b])

        def dep0(vec):
            m = lax.reduce_min(vec, (0,))
            return (m * 0.0).astype(jnp.int32)

        in_copy(i32(0), 0).start()

        def chunk_pair_body(pi, carry):
            for b in (0, 1):
                ci = pi * i32(2) + i32(b)
                in_copy(ci, b).wait()

                @pl.when(ci + i32(1) < i32(N_CHUNKS))
                def _():
                    in_copy(ci + i32(1), 1 - b).start()

                @pl.when(ci >= i32(2))
                def _():
                    out_copy(ci - i32(2), b).wait()

                indep = dep0(coords_v[b, 0, pl.ds(i32(0), 16)])
                compute_chunk(b, indep)
                outdep = dep0(feat_v[b, 0, pl.ds(i32(0), 16)])
                out_copy(ci, b, outdep).start()
            return carry

        def compute_chunk(b, indep):
            @functools.partial(
                plsc.parallel_loop, i32(0), i32(GROUPS), unroll=2)
            def group_body(g):
                o = g * i32(16) + indep
                dims = []
                for d in range(3):
                    xg = coords_v[b, d, pl.ds(o, 16)]
                    sc = xg * 0.1 + 0.5
                    sc = jnp.minimum(jnp.maximum(sc, 0.0), 1.0 - 1e-6)
                    xl = sc * nl_f
                    fi = xl.astype(jnp.int32)
                    w = xl - fi.astype(jnp.float32)
                    dims.append((fi, w))
                (fi0, w0), (fi1, w1), (fi2, w2) = dims
                a0 = (fi0 * c0, fi0 * c0 + c0)
                a1 = (fi1 * c1, fi1 * c1 + c1)
                a2 = (fi2 * c2, fi2 * c2 + c2)
                wx = (1.0 - w0, w0)
                wy = (1.0 - w1, w1)
                wz = (1.0 - w2, w2)
                wxy = {(i, j): wx[i] * wy[j] for i in (0, 1) for j in (0, 1)}
                acc0 = jnp.zeros((16,), jnp.float32)
                acc1 = jnp.zeros((16,), jnp.float32)
                for v in range(8):
                    b0, b1, b2 = (v >> 2) & 1, (v >> 1) & 1, v & 1
                    hidx = (a0[b0] ^ a1[b1] ^ a2[b2]) & (MAX_ENTRIES - 1)
                    didx = a0[b0] + a1[b1] + a2[b2]
                    idx = jnp.where(dense, didx, hidx)
                    f0 = idx * 2
                    g0 = plsc.load_gather(table_v, [f0])
                    g1 = plsc.load_gather(table_v, [f0 + 1])
                    vw = wxy[(b0, b1)] * wz[b2]
                    acc0 = acc0 + g0 * vw
                    acc1 = acc1 + g1 * vw
                feat_v[b, 0, pl.ds(o, 16)] = acc0
                feat_v[b, 1, pl.ds(o, 16)] = acc1

        lax.fori_loop(i32(0), i32(N_CHUNKS // 2), chunk_pair_body, i32(0))
        out_copy(i32(N_CHUNKS - 2), 0).wait()
        out_copy(i32(N_CHUNKS - 1), 1).wait()

    return enc_kernel(xT, tabs, consts)


_BB = 8192


def _mlp_body(enc_ref, xv_ref, wd0, bd0, wd1, bd1, wc0v, wc0h, bc0,
              wc1, bc1, wc2, bc2, out_ref):
    enc = enc_ref[...]
    mm = functools.partial(jnp.dot, preferred_element_type=jnp.float32)
    h = jnp.maximum(mm(wd0[...], enc) + bd0[...], 0.0)
    h = mm(wd1[...], h) + bd1[...]
    row = lax.broadcasted_iota(jnp.int32, (16, _BB), 0)
    hh = jnp.where(row == 0, jax.nn.sigmoid(h), jnp.maximum(h, 0.0))
    hc = mm(wc0h[...], hh) + mm(wc0v[...], xv_ref[...]) + bc0[...]
    hc = jnp.maximum(hc, 0.0)
    hc = jnp.maximum(mm(wc1[...], hc) + bc1[...], 0.0)
    hc = mm(wc2[...], hc) + bc2[...]
    out_ref[...] = jnp.concatenate(
        [hh[0:1, :], jax.nn.sigmoid(hc)], axis=0)


def _mlp_tc(enc, xvT, wd0T, bd0, wd1T, bd1, wc0vT, wc0hT, bc0,
            wc1T, bc1, wc2T, bc2):
    grid = B_TOTAL // _BB
    z = np.int32(0)
    full = lambda shape: pl.BlockSpec(shape, lambda i: (z, z))
    batch = lambda rows: pl.BlockSpec((rows, _BB), lambda i: (z, i))
    return pl.pallas_call(
        _mlp_body,
        grid=(grid,),
        in_specs=[
            batch(32),
            batch(3),
            full((64, 32)), full((64, 1)),
            full((16, 64)), full((16, 1)),
            full((64, 3)), full((64, 16)), full((64, 1)),
            full((64, 64)), full((64, 1)),
            full((3, 64)), full((3, 1)),
        ],
        out_specs=batch(4),
        out_shape=jax.ShapeDtypeStruct((4, B_TOTAL), jnp.float32),
    )(enc, xvT, wd0T, bd0, wd1T, bd1, wc0vT, wc0hT, bc0,
      wc1T, bc1, wc2T, bc2)


def kernel(x, tables, w_d0, b_d0, w_d1, b_d1, w_c0, b_c0, w_c1, b_c1,
           w_c2, b_c2):
    x = x.astype(jnp.float32)
    xT = x.T
    tabs = tables.astype(jnp.float32).reshape(NUM_LEVELS, 2 * MAX_ENTRIES)
    consts = jnp.asarray(_CONSTS.reshape(-1))

    enc = _encode_sc(xT[0:3], tabs, consts)

    outT = _mlp_tc(
        enc, xT[3:6],
        w_d0.T, b_d0.reshape(64, 1),
        w_d1.T, b_d1.reshape(16, 1),
        w_c0[16:19].T, w_c0[0:16].T, b_c0.reshape(64, 1),
        w_c1.T, b_c1.reshape(64, 1),
        w_c2.T, b_c2.reshape(3, 1),
    )
    return outT.T
